# Initial kernel scaffold; baseline (speedup 1.0000x reference)
import functools

import jax
import jax.numpy as jnp
from jax import lax
from jax.experimental import pallas as pl
from jax.experimental.pallas import tpu as pltpu
from jax.experimental.pallas import tpu_sc as plsc

N = 10000
D = 128
E = 320000

NC = 2
NS = 16
NW = NC * NS
EB = 128
NB = 80
E_PAD = NW * NB * EB
N_PAD = 10016
ROWS_PER_TILE = N_PAD // NS
DW = 16

BLK = 2504
GRID = N_PAD // BLK

_mesh = plsc.VectorSubcoreMesh(core_axis_name="c", subcore_axis_name="s")


@functools.partial(
    pl.kernel,
    out_type=jax.ShapeDtypeStruct((NC * N_PAD, DW), jnp.float32),
    mesh=_mesh,
    scratch_types=[
        pltpu.VMEM((EB,), jnp.int32),
        pltpu.VMEM((EB, DW), jnp.float32),
        pltpu.VMEM_SHARED((N_PAD, DW), jnp.float32),
    ],
)
def _deg_kernel(dst_hbm, z_hbm, out_hbm, didx, ones_v, dtab):
    cid = lax.axis_index("c")
    sid = lax.axis_index("s")
    wid = cid * NS + sid
    for j in range(EB):
        ones_v[j, :] = jnp.full((DW,), 1.0, jnp.float32)
    pltpu.sync_copy(z_hbm.at[pl.ds(sid * ROWS_PER_TILE, ROWS_PER_TILE)],
                    dtab.at[pl.ds(sid * ROWS_PER_TILE, ROWS_PER_TILE)])
    plsc.subcore_barrier()

    def body(g, carry):
        base = (wid * NB + g) * EB
        pltpu.sync_copy(dst_hbm.at[pl.ds(base, EB)], didx)
        pltpu.sync_copy(ones_v, dtab.at[didx], add=True)
        return carry

    lax.fori_loop(0, NB, body, 0)
    plsc.subcore_barrier()
    pltpu.sync_copy(dtab.at[pl.ds(sid * ROWS_PER_TILE, ROWS_PER_TILE)],
                    out_hbm.at[pl.ds(cid * N_PAD + sid * ROWS_PER_TILE,
                                     ROWS_PER_TILE)])


@functools.partial(
    pl.kernel,
    out_type=jax.ShapeDtypeStruct((NC * N_PAD, D), jnp.float32),
    mesh=_mesh,
    scratch_types=[
        pltpu.VMEM((EB,), jnp.int32),
        pltpu.VMEM((EB,), jnp.int32),
        pltpu.VMEM((EB, D), jnp.float32),
        pltpu.VMEM_SHARED((N_PAD, D), jnp.float32),
        pltpu.SemaphoreType.DMA,
    ],
)
def _agg_kernel(src_hbm, dst_hbm, hs_hbm, z_hbm, out_hbm,
                sidx, didx, rows, ytab, sem):
    cid = lax.axis_index("c")
    sid = lax.axis_index("s")
    wid = cid * NS + sid
    pltpu.sync_copy(z_hbm.at[pl.ds(sid * ROWS_PER_TILE, ROWS_PER_TILE)],
                    ytab.at[pl.ds(sid * ROWS_PER_TILE, ROWS_PER_TILE)])
    plsc.subcore_barrier()

    def body(g, carry):
        base = (wid * NB + g) * EB
        pltpu.sync_copy(src_hbm.at[pl.ds(base, EB)], sidx)
        pltpu.sync_copy(dst_hbm.at[pl.ds(base, EB)], didx)
        pltpu.async_copy(hs_hbm.at[sidx], rows, sem).wait()
        pltpu.sync_copy(rows, ytab.at[didx], add=True)
        return carry

    lax.fori_loop(0, NB, body, 0)
    plsc.subcore_barrier()
    pltpu.sync_copy(ytab.at[pl.ds(sid * ROWS_PER_TILE, ROWS_PER_TILE)],
                    out_hbm.at[pl.ds(cid * N_PAD + sid * ROWS_PER_TILE,
                                     ROWS_PER_TILE)])


def _ds_from_deg(deg_ref):
    deg = deg_ref[0, :, 0] + deg_ref[1, :, 0] + 1.0
    return lax.rsqrt(deg)


def _prep1_body(x_ref, w_ref, deg_ref, o_ref):
    h = jnp.dot(x_ref[...], w_ref[...],
                preferred_element_type=jnp.float32,
                precision=lax.Precision.HIGHEST)
    o_ref[...] = h * _ds_from_deg(deg_ref)[:, None]


def _mid_body(y_ref, hs_ref, deg_ref, b_ref, w_ref, o_ref):
    ds = _ds_from_deg(deg_ref)
    t = (y_ref[0] + y_ref[1] + hs_ref[...]) * ds[:, None] + b_ref[...]
    h1 = jnp.maximum(t, 0.0)
    h2 = jnp.dot(h1, w_ref[...],
                 preferred_element_type=jnp.float32,
                 precision=lax.Precision.HIGHEST)
    o_ref[...] = h2 * ds[:, None]


def _fin_body(y_ref, hs_ref, deg_ref, b_ref, g_ref, bt_ref, o_ref):
    ds = _ds_from_deg(deg_ref)
    t = (y_ref[0] + y_ref[1] + hs_ref[...]) * ds[:, None] + b_ref[...]
    mu = jnp.mean(t, axis=-1, keepdims=True)
    var = jnp.mean((t - mu) ** 2, axis=-1, keepdims=True)
    o_ref[...] = (t - mu) * lax.rsqrt(var + 1e-5) * g_ref[...] + bt_ref[...]


_row_spec = pl.BlockSpec((BLK, D), lambda i: (i, 0))
_deg_spec = pl.BlockSpec((2, BLK, DW), lambda i: (0, i, 0))
_y_spec = pl.BlockSpec((2, BLK, D), lambda i: (0, i, 0))
_mat_spec = pl.BlockSpec((D, D), lambda i: (0, 0))
_vec_spec = pl.BlockSpec((1, D), lambda i: (0, 0))
_out_shape = jax.ShapeDtypeStruct((N_PAD, D), jnp.float32)

_prep1 = pl.pallas_call(
    _prep1_body, grid=(GRID,),
    in_specs=[_row_spec, _mat_spec, _deg_spec],
    out_specs=_row_spec, out_shape=_out_shape)

_mid = pl.pallas_call(
    _mid_body, grid=(GRID,),
    in_specs=[_y_spec, _row_spec, _deg_spec, _vec_spec, _mat_spec],
    out_specs=_row_spec, out_shape=_out_shape)

_fin = pl.pallas_call(
    _fin_body, grid=(GRID,),
    in_specs=[_y_spec, _row_spec, _deg_spec, _vec_spec, _vec_spec, _vec_spec],
    out_specs=_row_spec, out_shape=_out_shape)


def kernel(x, edge_index, W1, b1, W2, b2, ln_gamma, ln_beta):
    pad_e = jnp.full((E_PAD - E,), N, jnp.int32)
    src = jnp.concatenate([edge_index[0], pad_e])
    dst = jnp.concatenate([edge_index[1], pad_e])
    x_pad = jnp.concatenate([x, jnp.zeros((N_PAD - N, D), jnp.float32)])
    z = jnp.zeros((N_PAD, D), jnp.float32)
    dz = jnp.zeros((N_PAD, DW), jnp.float32)

    deg2 = _deg_kernel(dst, dz).reshape(2, N_PAD, DW)
    hs1 = _prep1(x_pad, W1, deg2)
    y1 = _agg_kernel(src, dst, hs1, z).reshape(2, N_PAD, D)
    hs2 = _mid(y1, hs1, deg2, b1.reshape(1, D), W2)
    y2 = _agg_kernel(src, dst, hs2, z).reshape(2, N_PAD, D)
    out = _fin(y2, hs2, deg2, b2.reshape(1, D),
               ln_gamma.reshape(1, D), ln_beta.reshape(1, D))
    return out[:N]


# speedup vs baseline: 27.3786x; 27.3786x over previous
import functools

import jax
import jax.numpy as jnp
from jax import lax
from jax.experimental import pallas as pl
from jax.experimental.pallas import tpu as pltpu
from jax.experimental.pallas import tpu_sc as plsc

N = 10000
D = 128
E = 320000

NC = 2
NS = 16
NW = NC * NS
EB = 128
NB = 80
E_PAD = NW * NB * EB
N_PAD = 10112
ROWS_PER_TILE = N_PAD // NS
DW = 16

BLK = 2528
GRID = N_PAD // BLK

_mesh = plsc.VectorSubcoreMesh(core_axis_name="c", subcore_axis_name="s")


@functools.partial(
    pl.kernel,
    out_type=jax.ShapeDtypeStruct((NC * N_PAD, DW), jnp.float32),
    mesh=_mesh,
    scratch_types=[
        pltpu.VMEM((NB, EB), jnp.int32),
        pltpu.VMEM((EB, DW), jnp.float32),
        pltpu.VMEM_SHARED((N_PAD, DW), jnp.float32),
        pltpu.SemaphoreType.DMA,
    ],
)
def _deg_kernel(dst_hbm, z_hbm, out_hbm, didx, ones_v, dtab, sem):
    cid = lax.axis_index("c")
    sid = lax.axis_index("s")
    wid = cid * NS + sid
    for j in range(EB):
        ones_v[j, :] = jnp.full((DW,), 1.0, jnp.float32)
    pltpu.sync_copy(z_hbm.at[pl.ds(sid * ROWS_PER_TILE, ROWS_PER_TILE)],
                    dtab.at[pl.ds(sid * ROWS_PER_TILE, ROWS_PER_TILE)])
    pltpu.sync_copy(dst_hbm.at[pl.ds(wid * NB, NB)], didx)
    plsc.subcore_barrier()

    def body(g, carry):
        pltpu.sync_copy(ones_v, dtab.at[didx.at[g]], add=True)
        return carry

    lax.fori_loop(0, NB, body, 0)
    plsc.subcore_barrier()
    pltpu.sync_copy(dtab.at[pl.ds(sid * ROWS_PER_TILE, ROWS_PER_TILE)],
                    out_hbm.at[pl.ds(cid * N_PAD + sid * ROWS_PER_TILE,
                                     ROWS_PER_TILE)])


CH = 8
NCH = NB // CH


@functools.partial(
    pl.kernel,
    out_type=jax.ShapeDtypeStruct((NC * N_PAD, D), jnp.float32),
    mesh=_mesh,
    scratch_types=[
        pltpu.VMEM((2, CH, EB), jnp.int32),
        pltpu.VMEM((NB, EB), jnp.int32),
        pltpu.VMEM((2, EB, D), jnp.float32),
        pltpu.VMEM_SHARED((N_PAD, D), jnp.float32),
        [pltpu.SemaphoreType.DMA] * 2,
        [pltpu.SemaphoreType.DMA] * 2,
    ],
)
def _agg_kernel(src_hbm, dst_hbm, hs_hbm, z_hbm, out_hbm,
                sidx, didx, rows, ytab, gsems, isems):
    cid = lax.axis_index("c")
    sid = lax.axis_index("s")
    wid = cid * NS + sid
    base = wid * NB
    pltpu.sync_copy(z_hbm.at[pl.ds(sid * ROWS_PER_TILE, ROWS_PER_TILE)],
                    ytab.at[pl.ds(sid * ROWS_PER_TILE, ROWS_PER_TILE)])
    pltpu.sync_copy(dst_hbm.at[pl.ds(base, NB)], didx)
    pltpu.sync_copy(src_hbm.at[pl.ds(base, CH)], sidx.at[0])
    pltpu.async_copy(hs_hbm.at[sidx.at[0, 0]], rows.at[0], gsems[0])
    plsc.subcore_barrier()

    def chunk_pair(p, carry):
        for cc in range(2):
            c = 2 * p + cc
            @pl.when(c + 1 < NCH)
            def _():
                pltpu.async_copy(src_hbm.at[pl.ds(base + (c + 1) * CH, CH)],
                                 sidx.at[1 - cc], isems[1 - cc])

            for b in range(CH):
                g = c * CH + b
                kb = b % 2
                pltpu.make_async_copy(hs_hbm.at[sidx.at[cc, b]],
                                      rows.at[kb], gsems[kb]).wait()
                if b + 1 < CH:
                    pltpu.async_copy(hs_hbm.at[sidx.at[cc, b + 1]],
                                     rows.at[1 - kb], gsems[1 - kb])
                else:
                    @pl.when(c + 1 < NCH)
                    def _():
                        pltpu.make_async_copy(
                            src_hbm.at[pl.ds(base + (c + 1) * CH, CH)],
                            sidx.at[1 - cc], isems[1 - cc]).wait()
                        pltpu.async_copy(hs_hbm.at[sidx.at[1 - cc, 0]],
                                         rows.at[1 - kb], gsems[1 - kb])
                pltpu.sync_copy(rows.at[kb], ytab.at[didx.at[g]], add=True)
        return carry

    lax.fori_loop(0, NCH // 2, chunk_pair, 0)
    plsc.subcore_barrier()
    pltpu.sync_copy(ytab.at[pl.ds(sid * ROWS_PER_TILE, ROWS_PER_TILE)],
                    out_hbm.at[pl.ds(cid * N_PAD + sid * ROWS_PER_TILE,
                                     ROWS_PER_TILE)])


def _ds_from_deg(deg_ref):
    deg = deg_ref[0, :, 0] + deg_ref[1, :, 0] + 1.0
    return lax.rsqrt(deg)


def _prep1_body(x_ref, w_ref, deg_ref, o_ref):
    h = jnp.dot(x_ref[...], w_ref[...],
                preferred_element_type=jnp.float32,
                precision=lax.Precision.HIGHEST)
    o_ref[...] = h * _ds_from_deg(deg_ref)[:, None]


def _mid_body(y_ref, hs_ref, deg_ref, b_ref, w_ref, o_ref):
    ds = _ds_from_deg(deg_ref)
    t = (y_ref[0] + y_ref[1] + hs_ref[...]) * ds[:, None] + b_ref[...]
    h1 = jnp.maximum(t, 0.0)
    h2 = jnp.dot(h1, w_ref[...],
                 preferred_element_type=jnp.float32,
                 precision=lax.Precision.HIGHEST)
    o_ref[...] = h2 * ds[:, None]


def _fin_body(y_ref, hs_ref, deg_ref, b_ref, g_ref, bt_ref, o_ref):
    ds = _ds_from_deg(deg_ref)
    t = (y_ref[0] + y_ref[1] + hs_ref[...]) * ds[:, None] + b_ref[...]
    mu = jnp.mean(t, axis=-1, keepdims=True)
    var = jnp.mean((t - mu) ** 2, axis=-1, keepdims=True)
    o_ref[...] = (t - mu) * lax.rsqrt(var + 1e-5) * g_ref[...] + bt_ref[...]


_row_spec = pl.BlockSpec((BLK, D), lambda i: (i, 0))
_deg_spec = pl.BlockSpec((2, BLK, DW), lambda i: (0, i, 0))
_y_spec = pl.BlockSpec((2, BLK, D), lambda i: (0, i, 0))
_mat_spec = pl.BlockSpec((D, D), lambda i: (0, 0))
_vec_spec = pl.BlockSpec((1, D), lambda i: (0, 0))
_out_shape = jax.ShapeDtypeStruct((N_PAD, D), jnp.float32)

_prep1 = pl.pallas_call(
    _prep1_body, grid=(GRID,),
    in_specs=[_row_spec, _mat_spec, _deg_spec],
    out_specs=_row_spec, out_shape=_out_shape)

_mid = pl.pallas_call(
    _mid_body, grid=(GRID,),
    in_specs=[_y_spec, _row_spec, _deg_spec, _vec_spec, _mat_spec],
    out_specs=_row_spec, out_shape=_out_shape)

_fin = pl.pallas_call(
    _fin_body, grid=(GRID,),
    in_specs=[_y_spec, _row_spec, _deg_spec, _vec_spec, _vec_spec, _vec_spec],
    out_specs=_row_spec, out_shape=_out_shape)


def kernel(x, edge_index, W1, b1, W2, b2, ln_gamma, ln_beta):
    n_pad_e = E_PAD - E
    pad_rows = (N + jnp.arange(n_pad_e, dtype=jnp.int32) % (N_PAD - N)
                ).reshape(NW, n_pad_e // NW)

    def _shard(e):
        return jnp.concatenate(
            [e.reshape(NW, E // NW), pad_rows], axis=1).reshape(E_PAD // EB, EB)

    src = _shard(edge_index[0])
    dst = _shard(edge_index[1])
    x_pad = jnp.concatenate([x, jnp.zeros((N_PAD - N, D), jnp.float32)])
    z = jnp.zeros((N_PAD, D), jnp.float32)
    dz = jnp.zeros((N_PAD, DW), jnp.float32)

    deg2 = _deg_kernel(dst, dz).reshape(2, N_PAD, DW)
    hs1 = _prep1(x_pad, W1, deg2)
    y1 = _agg_kernel(src, dst, hs1, z).reshape(2, N_PAD, D)
    hs2 = _mid(y1, hs1, deg2, b1.reshape(1, D), W2)
    y2 = _agg_kernel(src, dst, hs2, z).reshape(2, N_PAD, D)
    out = _fin(y2, hs2, deg2, b2.reshape(1, D),
               ln_gamma.reshape(1, D), ln_beta.reshape(1, D))
    return out[:N]


# speedup vs baseline: 28.4439x; 1.0389x over previous
import functools

import jax
import jax.numpy as jnp
from jax import lax
from jax.experimental import pallas as pl
from jax.experimental.pallas import tpu as pltpu
from jax.experimental.pallas import tpu_sc as plsc

N = 10000
D = 128
E = 320000

NC = 2
NS = 16
NW = NC * NS
EB = 128
NB = 80
E_PAD = NW * NB * EB
N_PAD = 10112
ROWS_PER_TILE = N_PAD // NS
DW = 16

BLK = 2528
GRID = N_PAD // BLK

_mesh = plsc.VectorSubcoreMesh(core_axis_name="c", subcore_axis_name="s")


@functools.partial(
    pl.kernel,
    out_type=jax.ShapeDtypeStruct((NC * N_PAD, DW), jnp.float32),
    mesh=_mesh,
    scratch_types=[
        pltpu.VMEM((NB, EB), jnp.int32),
        pltpu.VMEM((EB, DW), jnp.float32),
        pltpu.VMEM_SHARED((N_PAD, DW), jnp.float32),
        pltpu.SemaphoreType.DMA,
    ],
)
def _deg_kernel(dst_hbm, z_hbm, out_hbm, didx, ones_v, dtab, sem):
    cid = lax.axis_index("c")
    sid = lax.axis_index("s")
    wid = cid * NS + sid
    for j in range(EB):
        ones_v[j, :] = jnp.full((DW,), 1.0, jnp.float32)
    pltpu.sync_copy(z_hbm.at[pl.ds(sid * ROWS_PER_TILE, ROWS_PER_TILE)],
                    dtab.at[pl.ds(sid * ROWS_PER_TILE, ROWS_PER_TILE)])
    pltpu.sync_copy(dst_hbm.at[pl.ds(wid * NB, NB)], didx)
    plsc.subcore_barrier()

    def body(g, carry):
        pltpu.sync_copy(ones_v, dtab.at[didx.at[g]], add=True)
        return carry

    lax.fori_loop(0, NB, body, 0)
    plsc.subcore_barrier()
    pltpu.sync_copy(dtab.at[pl.ds(sid * ROWS_PER_TILE, ROWS_PER_TILE)],
                    out_hbm.at[pl.ds(cid * N_PAD + sid * ROWS_PER_TILE,
                                     ROWS_PER_TILE)])


CH = 8
NCH = NB // CH
EH = EB // 2


def _fire_gather(hs_hbm, idx_row, rows, k, gsems, hsems):
    pltpu.async_copy(hs_hbm.at[idx_row.at[pl.ds(0, EH)]],
                     rows.at[k, pl.ds(0, EH)], gsems[k])
    pltpu.async_copy(hs_hbm.at[idx_row.at[pl.ds(EH, EH)]],
                     rows.at[k, pl.ds(EH, EH)], hsems[k])


def _wait_gather(hs_hbm, idx_row, rows, k, gsems, hsems):
    pltpu.make_async_copy(hs_hbm.at[idx_row.at[pl.ds(0, EH)]],
                          rows.at[k, pl.ds(0, EH)], gsems[k]).wait()
    pltpu.make_async_copy(hs_hbm.at[idx_row.at[pl.ds(EH, EH)]],
                          rows.at[k, pl.ds(EH, EH)], hsems[k]).wait()


@functools.partial(
    pl.kernel,
    out_type=jax.ShapeDtypeStruct((NC * N_PAD, D), jnp.float32),
    mesh=_mesh,
    scratch_types=[
        pltpu.VMEM((2, CH, EB), jnp.int32),
        pltpu.VMEM((NB, EB), jnp.int32),
        pltpu.VMEM((2, EB, D), jnp.float32),
        pltpu.VMEM_SHARED((N_PAD, D), jnp.float32),
        [pltpu.SemaphoreType.DMA] * 2,
        [pltpu.SemaphoreType.DMA] * 2,
        [pltpu.SemaphoreType.DMA] * 2,
    ],
)
def _agg_kernel(src_hbm, dst_hbm, hs_hbm, z_hbm, out_hbm,
                sidx, didx, rows, ytab, gsems, hsems, isems):
    cid = lax.axis_index("c")
    sid = lax.axis_index("s")
    wid = cid * NS + sid
    base = wid * NB
    pltpu.sync_copy(z_hbm.at[pl.ds(sid * ROWS_PER_TILE, ROWS_PER_TILE)],
                    ytab.at[pl.ds(sid * ROWS_PER_TILE, ROWS_PER_TILE)])
    pltpu.sync_copy(dst_hbm.at[pl.ds(base, NB)], didx)
    pltpu.sync_copy(src_hbm.at[pl.ds(base, CH)], sidx.at[0])
    _fire_gather(hs_hbm, sidx.at[0, 0], rows, 0, gsems, hsems)
    plsc.subcore_barrier()

    def chunk_pair(p, carry):
        for cc in range(2):
            c = 2 * p + cc
            @pl.when(c + 1 < NCH)
            def _():
                pltpu.async_copy(src_hbm.at[pl.ds(base + (c + 1) * CH, CH)],
                                 sidx.at[1 - cc], isems[1 - cc])

            for b in range(CH):
                g = c * CH + b
                kb = b % 2
                _wait_gather(hs_hbm, sidx.at[cc, b], rows, kb, gsems, hsems)
                if b + 1 < CH:
                    _fire_gather(hs_hbm, sidx.at[cc, b + 1], rows, 1 - kb,
                                 gsems, hsems)
                else:
                    @pl.when(c + 1 < NCH)
                    def _():
                        pltpu.make_async_copy(
                            src_hbm.at[pl.ds(base + (c + 1) * CH, CH)],
                            sidx.at[1 - cc], isems[1 - cc]).wait()
                        _fire_gather(hs_hbm, sidx.at[1 - cc, 0], rows, 1 - kb,
                                     gsems, hsems)
                pltpu.sync_copy(rows.at[kb], ytab.at[didx.at[g]], add=True)
        return carry

    lax.fori_loop(0, NCH // 2, chunk_pair, 0)
    plsc.subcore_barrier()
    pltpu.sync_copy(ytab.at[pl.ds(sid * ROWS_PER_TILE, ROWS_PER_TILE)],
                    out_hbm.at[pl.ds(cid * N_PAD + sid * ROWS_PER_TILE,
                                     ROWS_PER_TILE)])


def _ds_from_deg(deg_ref):
    deg = deg_ref[0, :, 0] + deg_ref[1, :, 0] + 1.0
    return lax.rsqrt(deg)


def _prep1_body(x_ref, w_ref, deg_ref, o_ref):
    h = jnp.dot(x_ref[...], w_ref[...],
                preferred_element_type=jnp.float32,
                precision=lax.Precision.HIGHEST)
    o_ref[...] = h * _ds_from_deg(deg_ref)[:, None]


def _mid_body(y_ref, hs_ref, deg_ref, b_ref, w_ref, o_ref):
    ds = _ds_from_deg(deg_ref)
    t = (y_ref[0] + y_ref[1] + hs_ref[...]) * ds[:, None] + b_ref[...]
    h1 = jnp.maximum(t, 0.0)
    h2 = jnp.dot(h1, w_ref[...],
                 preferred_element_type=jnp.float32,
                 precision=lax.Precision.HIGHEST)
    o_ref[...] = h2 * ds[:, None]


def _fin_body(y_ref, hs_ref, deg_ref, b_ref, g_ref, bt_ref, o_ref):
    ds = _ds_from_deg(deg_ref)
    t = (y_ref[0] + y_ref[1] + hs_ref[...]) * ds[:, None] + b_ref[...]
    mu = jnp.mean(t, axis=-1, keepdims=True)
    var = jnp.mean((t - mu) ** 2, axis=-1, keepdims=True)
    o_ref[...] = (t - mu) * lax.rsqrt(var + 1e-5) * g_ref[...] + bt_ref[...]


_row_spec = pl.BlockSpec((BLK, D), lambda i: (i, 0))
_deg_spec = pl.BlockSpec((2, BLK, DW), lambda i: (0, i, 0))
_y_spec = pl.BlockSpec((2, BLK, D), lambda i: (0, i, 0))
_mat_spec = pl.BlockSpec((D, D), lambda i: (0, 0))
_vec_spec = pl.BlockSpec((1, D), lambda i: (0, 0))
_out_shape = jax.ShapeDtypeStruct((N_PAD, D), jnp.float32)

_prep1 = pl.pallas_call(
    _prep1_body, grid=(GRID,),
    in_specs=[_row_spec, _mat_spec, _deg_spec],
    out_specs=_row_spec, out_shape=_out_shape)

_mid = pl.pallas_call(
    _mid_body, grid=(GRID,),
    in_specs=[_y_spec, _row_spec, _deg_spec, _vec_spec, _mat_spec],
    out_specs=_row_spec, out_shape=_out_shape)

FBLK = 2000

_fin = pl.pallas_call(
    _fin_body, grid=(N // FBLK,),
    in_specs=[pl.BlockSpec((2, FBLK, D), lambda i: (0, i, 0)),
              pl.BlockSpec((FBLK, D), lambda i: (i, 0)),
              pl.BlockSpec((2, FBLK, DW), lambda i: (0, i, 0)),
              _vec_spec, _vec_spec, _vec_spec],
    out_specs=pl.BlockSpec((FBLK, D), lambda i: (i, 0)),
    out_shape=jax.ShapeDtypeStruct((N, D), jnp.float32))


def kernel(x, edge_index, W1, b1, W2, b2, ln_gamma, ln_beta):
    pad_rows = N + jnp.arange(E_PAD - E, dtype=jnp.int32) % (N_PAD - N)
    src = jnp.concatenate([edge_index[0], pad_rows]).reshape(E_PAD // EB, EB)
    dst = jnp.concatenate([edge_index[1], pad_rows]).reshape(E_PAD // EB, EB)
    x_pad = jnp.concatenate([x, jnp.zeros((N_PAD - N, D), jnp.float32)])
    z = jnp.zeros((N_PAD, D), jnp.float32)
    dz = jnp.zeros((N_PAD, DW), jnp.float32)

    deg2 = _deg_kernel(dst, dz).reshape(2, N_PAD, DW)
    hs1 = _prep1(x_pad, W1, deg2)
    y1 = _agg_kernel(src, dst, hs1, z).reshape(2, N_PAD, D)
    hs2 = _mid(y1, hs1, deg2, b1.reshape(1, D), W2)
    y2 = _agg_kernel(src, dst, hs2, z).reshape(2, N_PAD, D)
    return _fin(y2, hs2, deg2, b2.reshape(1, D),
                ln_gamma.reshape(1, D), ln_beta.reshape(1, D))
